# baseline (speedup 1.0000x reference)
import functools

import jax
import jax.numpy as jnp
from jax import lax
from jax.experimental import pallas as pl
from jax.experimental.pallas import tpu as pltpu
from jax.experimental.pallas import tpu_sc as plsc


def _make_gather(v, d, b_tot, nw, n_chunk, chunk):
    b_per_w = b_tot // nw
    assert b_per_w == n_chunk * chunk

    mesh = plsc.VectorSubcoreMesh(core_axis_name="c", subcore_axis_name="s")

    @functools.partial(
        pl.kernel,
        mesh=mesh,
        out_type=jax.ShapeDtypeStruct((b_tot, d), jnp.float32),
        compiler_params=pltpu.CompilerParams(use_tc_tiling_on_sc=False),
        scratch_types=[
            pltpu.VMEM((b_per_w,), jnp.int32),
            pltpu.VMEM((chunk, d), jnp.float32),
            pltpu.VMEM((chunk, d), jnp.float32),
            pltpu.SemaphoreType.DMA,
            pltpu.SemaphoreType.DMA,
        ],
    )
    def gather_kernel(table_hbm, idx_hbm, out_hbm, idx_v, rows0, rows1, sem0, sem1):
        nc = plsc.get_sparse_core_info().num_cores
        wid = lax.axis_index("s") * nc + lax.axis_index("c")
        base = wid * b_per_w
        pltpu.sync_copy(idx_hbm.at[wid], idx_v)

        bufs = (rows0, rows1)
        sems = (sem0, sem1)
        copies = [
            pltpu.async_copy(
                table_hbm.at[idx_v.at[pl.ds(0, chunk)]], bufs[0], sems[0]
            )
        ]
        for j in range(n_chunk):
            b = j % 2
            copies[j].wait()
            if j + 1 < n_chunk:
                nb = (j + 1) % 2
                copies.append(
                    pltpu.async_copy(
                        table_hbm.at[idx_v.at[pl.ds((j + 1) * chunk, chunk)]],
                        bufs[nb],
                        sems[nb],
                    )
                )
            pltpu.sync_copy(bufs[b], out_hbm.at[pl.ds(base + j * chunk, chunk)])

    return gather_kernel


def kernel(title_text, title_mask, word_embedding):
    batch, title_len = title_text.shape
    v, d = word_embedding.shape
    b_tot = batch * title_len

    info = plsc.get_sparse_core_info()
    nw = info.num_cores * info.num_subcores
    b_per_w = b_tot // nw
    chunk = 128
    n_chunk = b_per_w // chunk

    gather = _make_gather(v, d, b_tot, nw, n_chunk, chunk)
    idx = title_text.reshape(nw, b_per_w)
    out = gather(word_embedding, idx)
    return out.reshape(batch, title_len, d)
